# baseline (device time: 172850 ns/iter reference)
import jax
import jax.numpy as jnp
from jax import lax
from jax.experimental import pallas as pl
from jax.experimental.pallas import tpu as pltpu

N_DEV = 4


def kernel(x, w_mat, scale_x, scale_w):
    m, k_per = x.shape
    _, n = w_mat.shape
    m_per = m // N_DEV
    scale = (scale_x * scale_w).reshape(1, 1)

    def body(x_ref, w_ref, scale_ref, out_ref,
             send_buf, recv_buf, send_sems, recv_sems):
        my = lax.axis_index("i")
        left = (my + N_DEV - 1) % N_DEV
        right = (my + 1) % N_DEV

        barrier = pltpu.get_barrier_semaphore()
        for nbr in (left, right):
            pl.semaphore_signal(
                barrier, inc=1,
                device_id=(nbr,), device_id_type=pl.DeviceIdType.MESH,
            )
        pl.semaphore_wait(barrier, 2)

        def partial(c):
            xs = x_ref[pl.ds(c * m_per, m_per), :]
            return lax.dot_general(
                xs, w_ref[:, :],
                (((1,), (0,)), ((), ())),
                preferred_element_type=jnp.int32,
            )

        for t in range(N_DEV - 1):
            c = (my + N_DEV - 1 - t) % N_DEV
            p = partial(c).astype(jnp.float32)
            if t == 0:
                acc = p
            else:
                acc = recv_buf[t - 1].astype(jnp.float32) + p
            send_buf[:, :] = acc.astype(jnp.bfloat16)
            rdma = pltpu.make_async_remote_copy(
                src_ref=send_buf,
                dst_ref=recv_buf.at[t],
                send_sem=send_sems.at[t],
                recv_sem=recv_sems.at[t],
                device_id=(right,),
                device_id_type=pl.DeviceIdType.MESH,
            )
            rdma.start()
            rdma.wait()

        acc = recv_buf[N_DEV - 2].astype(jnp.float32) + \
            partial(my).astype(jnp.float32)
        y = acc * scale_ref[0, 0]
        out_ref[:, :] = y / (1.0 + jnp.exp(-jnp.clip(y, -60.0, 60.0)))

    return pl.pallas_call(
        body,
        out_shape=jax.ShapeDtypeStruct((m_per, n), jnp.float32),
        in_specs=[
            pl.BlockSpec(memory_space=pltpu.VMEM),
            pl.BlockSpec(memory_space=pltpu.VMEM),
            pl.BlockSpec(memory_space=pltpu.SMEM),
        ],
        out_specs=pl.BlockSpec(memory_space=pltpu.VMEM),
        scratch_shapes=[
            pltpu.VMEM((m_per, n), jnp.bfloat16),
            pltpu.VMEM((N_DEV - 1, m_per, n), jnp.bfloat16),
            pltpu.SemaphoreType.DMA((N_DEV - 1,)),
            pltpu.SemaphoreType.DMA((N_DEV - 1,)),
        ],
        compiler_params=pltpu.CompilerParams(collective_id=0),
    )(x, w_mat, scale)


# device time: 100398 ns/iter; 1.7216x vs baseline; 1.7216x over previous
import jax
import jax.numpy as jnp
from jax import lax
from jax.experimental import pallas as pl
from jax.experimental.pallas import tpu as pltpu

N_DEV = 4
N_HOP = N_DEV - 1


def kernel(x, w_mat, scale_x, scale_w):
    m, k_per = x.shape
    _, n = w_mat.shape
    m_per = m // N_DEV
    n_half = n // 2
    scale = (scale_x * scale_w).reshape(1, 1)

    def body(x_ref, w_ref, scale_ref, out_ref,
             send_r, send_l, recv_r, recv_l,
             ssem_r, rsem_r, ssem_l, rsem_l):
        my = lax.axis_index("i")
        left = (my + N_DEV - 1) % N_DEV
        right = (my + 1) % N_DEV

        barrier = pltpu.get_barrier_semaphore()
        for nbr in (left, right):
            pl.semaphore_signal(
                barrier, inc=1,
                device_id=(nbr,), device_id_type=pl.DeviceIdType.MESH,
            )
        pl.semaphore_wait(barrier, 2)

        def partial_r(c):
            return lax.dot_general(
                x_ref[pl.ds(c * m_per, m_per), :], w_ref[:, :n_half],
                (((1,), (0,)), ((), ())),
                preferred_element_type=jnp.int32,
            )

        def partial_l(c):
            return lax.dot_general(
                x_ref[pl.ds(c * m_per, m_per), :], w_ref[:, n_half:],
                (((1,), (0,)), ((), ())),
                preferred_element_type=jnp.int32,
            )

        def make_rdma(t, direction):
            if direction == "r":
                return pltpu.make_async_remote_copy(
                    src_ref=send_r.at[t], dst_ref=recv_r.at[t],
                    send_sem=ssem_r.at[t], recv_sem=rsem_r.at[t],
                    device_id=(right,), device_id_type=pl.DeviceIdType.MESH,
                )
            return pltpu.make_async_remote_copy(
                src_ref=send_l.at[t], dst_ref=recv_l.at[t],
                send_sem=ssem_l.at[t], recv_sem=rsem_l.at[t],
                device_id=(left,), device_id_type=pl.DeviceIdType.MESH,
            )

        rdmas = []
        for t in range(N_HOP):
            cr = (my + N_DEV - 1 - t) % N_DEV
            cl = (my + 1 + t) % N_DEV
            pr = partial_r(cr).astype(jnp.float32)
            pll = partial_l(cl).astype(jnp.float32)
            if t > 0:
                rdmas[t - 1][0].wait_recv()
                rdmas[t - 1][1].wait_recv()
                pr = pr + recv_r[t - 1].astype(jnp.float32)
                pll = pll + recv_l[t - 1].astype(jnp.float32)
            send_r[t, :, :] = pr.astype(jnp.bfloat16)
            send_l[t, :, :] = pll.astype(jnp.bfloat16)
            rr, rl = make_rdma(t, "r"), make_rdma(t, "l")
            rr.start()
            rl.start()
            rdmas.append((rr, rl))

        pr = partial_r(my).astype(jnp.float32)
        pll = partial_l(my).astype(jnp.float32)
        rdmas[N_HOP - 1][0].wait_recv()
        rdmas[N_HOP - 1][1].wait_recv()
        acc_r = recv_r[N_HOP - 1].astype(jnp.float32) + pr
        acc_l = recv_l[N_HOP - 1].astype(jnp.float32) + pll
        s = scale_ref[0, 0]
        yr = acc_r * s
        yl = acc_l * s
        out_ref[:, :n_half] = yr / (1.0 + jnp.exp(-jnp.clip(yr, -60.0, 60.0)))
        out_ref[:, n_half:] = yl / (1.0 + jnp.exp(-jnp.clip(yl, -60.0, 60.0)))

        for rr, rl in rdmas:
            rr.wait_send()
            rl.wait_send()

    return pl.pallas_call(
        body,
        out_shape=jax.ShapeDtypeStruct((m_per, n), jnp.float32),
        in_specs=[
            pl.BlockSpec(memory_space=pltpu.VMEM),
            pl.BlockSpec(memory_space=pltpu.VMEM),
            pl.BlockSpec(memory_space=pltpu.SMEM),
        ],
        out_specs=pl.BlockSpec(memory_space=pltpu.VMEM),
        scratch_shapes=[
            pltpu.VMEM((N_HOP, m_per, n_half), jnp.bfloat16),
            pltpu.VMEM((N_HOP, m_per, n_half), jnp.bfloat16),
            pltpu.VMEM((N_HOP, m_per, n_half), jnp.bfloat16),
            pltpu.VMEM((N_HOP, m_per, n_half), jnp.bfloat16),
            pltpu.SemaphoreType.DMA((N_HOP,)),
            pltpu.SemaphoreType.DMA((N_HOP,)),
            pltpu.SemaphoreType.DMA((N_HOP,)),
            pltpu.SemaphoreType.DMA((N_HOP,)),
        ],
        compiler_params=pltpu.CompilerParams(
            collective_id=0,
            vmem_limit_bytes=64 * 1024 * 1024,
        ),
    )(x, w_mat, scale)
